# baseline (speedup 1.0000x reference)
import functools

import jax
import jax.numpy as jnp
import numpy as np
from jax import lax
from jax.experimental import pallas as pl
from jax.experimental.pallas import tpu as pltpu
from jax.experimental.pallas import tpu_sc as plsc

VOCAB = 1000000
EMBED = 64
SEQ_LEN = 200
N_SEG = 3
NB = 4096

NC = 2
NS = 16
F_PER_CORE = EMBED // NC
TILE_N = SEQ_LEN * 2 * 128
NCH = 8
CH_N = TILE_N // NCH


def _make_pe(max_len, d):
    position = jnp.arange(max_len, dtype=jnp.float32)[:, None]
    div_term = jnp.exp(
        jnp.arange(0, d, 2, dtype=jnp.float32) * (-np.log(10000.0) / d)
    )
    pe = jnp.zeros((max_len, d), dtype=jnp.float32)
    pe = pe.at[:, 0::2].set(jnp.sin(position * div_term))
    pe = pe.at[:, 1::2].set(jnp.cos(position * div_term))
    return pe


@functools.partial(
    pl.kernel,
    out_type=jax.ShapeDtypeStruct((EMBED, NS * TILE_N), jnp.float32),
    mesh=plsc.VectorSubcoreMesh(core_axis_name="c", subcore_axis_name="s"),
    scratch_types=[
        pltpu.VMEM((TILE_N,), jnp.int32),
        pltpu.VMEM((CH_N,), jnp.float32),
        pltpu.VMEM((CH_N,), jnp.float32),
        pltpu.VMEM_SHARED((VOCAB,), jnp.float32),
        pltpu.SemaphoreType.DMA,
        pltpu.SemaphoreType.DMA,
        pltpu.SemaphoreType.DMA,
        pltpu.SemaphoreType.DMA,
    ],
    compiler_params=pltpu.CompilerParams(
        use_tc_tiling_on_sc=False,
        internal_scratch_in_bytes=128 * 1024,
    ),
)
def _sc_gather(seq_hbm, tok_hbm, out_hbm,
               idx_v, ob0, ob1, spb, ssem, gsem, wsem0, wsem1):
    c = lax.axis_index("c")
    s = lax.axis_index("s")
    fbase = F_PER_CORE * c

    pltpu.sync_copy(seq_hbm.at[s], idx_v)

    obs = (ob0, ob1)
    wsems = (wsem0, wsem1)

    def floop(fl, _):
        f = fbase + fl

        plsc.subcore_barrier()

        @pl.when(s == 0)
        def _():
            pltpu.async_copy(tok_hbm.at[f], spb, ssem).wait()

        plsc.subcore_barrier()

        for h in range(NCH):
            ob = obs[h % 2]
            dst = out_hbm.at[f, pl.ds(s * TILE_N + h * CH_N, CH_N)]

            @pl.when(fl * NCH + h >= 2)
            def _():
                pltpu.make_async_copy(ob, dst, wsems[h % 2]).wait()

            pltpu.async_copy(
                spb.at[idx_v.at[pl.ds(h * CH_N, CH_N)]], ob, gsem
            ).wait()
            pltpu.async_copy(ob, dst, wsems[h % 2])
        return 0

    lax.fori_loop(0, F_PER_CORE, floop, 0)

    flast = fbase + F_PER_CORE - 1
    for h in range(NCH - 2, NCH):
        dst = out_hbm.at[flast, pl.ds(s * TILE_N + h * CH_N, CH_N)]
        pltpu.make_async_copy(obs[h % 2], dst, wsems[h % 2]).wait()


TCHUNK = 8
NCOL = TCHUNK * 2 * 128


def _tc_add_body(g_ref, seg_ref, comb_ref, o_ref):
    for tt in range(TCHUNK):
        for j2 in range(2):
            segrow = seg_ref[j2, tt, :][None, :]
            m0 = segrow == 0
            m1 = segrow == 1
            col = (tt * 2 + j2) * 128
            for fo in range(8):
                a0 = comb_ref[0, 0, pl.ds(fo * 8, 8), tt][:, None]
                a1 = comb_ref[1, 0, pl.ds(fo * 8, 8), tt][:, None]
                a2 = comb_ref[2, 0, pl.ds(fo * 8, 8), tt][:, None]
                add = jnp.where(m0, a0, jnp.where(m1, a1, a2))
                o_ref[tt, fo, j2, :, :] = (
                    g_ref[pl.ds(fo * 8, 8), pl.ds(col, 128)] + add
                )


def _tc_add(gathered, segP, combP):
    return pl.pallas_call(
        _tc_add_body,
        grid=(EMBED * NS * TILE_N // (EMBED * NCOL),),
        in_specs=[
            pl.BlockSpec((EMBED, NCOL), lambda j: (0, j)),
            pl.BlockSpec((2, TCHUNK, 128), lambda j: (j // 25, j % 25, 0)),
            pl.BlockSpec((N_SEG, 1, EMBED, 128), lambda j: (0, j % 25, 0, 0)),
        ],
        out_specs=pl.BlockSpec((TCHUNK, 8, 2, 8, 128),
                               lambda j: (j % 25, 0, j // 25, 0, 0)),
        out_shape=jax.ShapeDtypeStruct((SEQ_LEN, 8, NB // 128, 8, 128),
                                       jnp.float32),
    )(gathered, segP, combP)


def kernel(sequence, segment_label, token_table, segment_table):
    pe = _make_pe(512, EMBED)[:SEQ_LEN]
    comb3 = segment_table[:, None, :] + pe[None, :, :]
    combP = jnp.pad(
        comb3.reshape(N_SEG, 25, TCHUNK, EMBED).transpose(0, 1, 3, 2),
        ((0, 0), (0, 0), (0, 0), (0, 128 - TCHUNK)),
    )
    seqP = (sequence.T.astype(jnp.int32)
            .reshape(SEQ_LEN, NS, 2, 128)
            .transpose(1, 0, 2, 3)
            .reshape(NS, TILE_N))
    segP = (segment_label.T.astype(jnp.int32)
            .reshape(SEQ_LEN, NB // 128, 128)
            .transpose(1, 0, 2))
    tokT = token_table.T
    gathered = _sc_gather(seqP, tokT)
    out5 = _tc_add(gathered, segP, combP)
    return out5.transpose(2, 4, 0, 1, 3).reshape(NB, SEQ_LEN, EMBED)


# speedup vs baseline: 1.0055x; 1.0055x over previous
import functools

import jax
import jax.numpy as jnp
import numpy as np
from jax import lax
from jax.experimental import pallas as pl
from jax.experimental.pallas import tpu as pltpu
from jax.experimental.pallas import tpu_sc as plsc

VOCAB = 1000000
EMBED = 64
SEQ_LEN = 200
N_SEG = 3
NB = 4096

NC = 2
NS = 16
F_PER_CORE = EMBED // NC
TILE_N = SEQ_LEN * 2 * 128
NCH = 8
CH_N = TILE_N // NCH


def _make_pe(max_len, d):
    position = jnp.arange(max_len, dtype=jnp.float32)[:, None]
    div_term = jnp.exp(
        jnp.arange(0, d, 2, dtype=jnp.float32) * (-np.log(10000.0) / d)
    )
    pe = jnp.zeros((max_len, d), dtype=jnp.float32)
    pe = pe.at[:, 0::2].set(jnp.sin(position * div_term))
    pe = pe.at[:, 1::2].set(jnp.cos(position * div_term))
    return pe


@functools.partial(
    pl.kernel,
    out_type=jax.ShapeDtypeStruct((EMBED, NS * TILE_N), jnp.float32),
    mesh=plsc.VectorSubcoreMesh(core_axis_name="c", subcore_axis_name="s"),
    scratch_types=[
        pltpu.VMEM((TILE_N,), jnp.int32),
        pltpu.VMEM((CH_N,), jnp.float32),
        pltpu.VMEM((CH_N,), jnp.float32),
        pltpu.VMEM_SHARED((VOCAB,), jnp.float32),
        pltpu.SemaphoreType.DMA,
        pltpu.SemaphoreType.DMA,
        pltpu.SemaphoreType.DMA,
        pltpu.SemaphoreType.DMA,
    ],
    compiler_params=pltpu.CompilerParams(
        use_tc_tiling_on_sc=False,
        internal_scratch_in_bytes=128 * 1024,
    ),
)
def _sc_gather(seq_hbm, tok_hbm, out_hbm,
               idx_v, ob0, ob1, spb, ssem, gsem, wsem0, wsem1):
    c = lax.axis_index("c")
    s = lax.axis_index("s")
    fbase = F_PER_CORE * c

    pltpu.sync_copy(seq_hbm.at[s], idx_v)

    obs = (ob0, ob1)
    wsems = (wsem0, wsem1)

    def floop(fl, _):
        f = fbase + fl

        plsc.subcore_barrier()

        @pl.when(s == 0)
        def _():
            pltpu.async_copy(tok_hbm.at[f], spb, ssem).wait()

        plsc.subcore_barrier()

        for h in range(NCH):
            ob = obs[h % 2]
            dst = out_hbm.at[f, pl.ds(s * TILE_N + h * CH_N, CH_N)]

            @pl.when(fl * NCH + h >= 2)
            def _():
                pltpu.make_async_copy(ob, dst, wsems[h % 2]).wait()

            pltpu.async_copy(
                spb.at[idx_v.at[pl.ds(h * CH_N, CH_N)]], ob, gsem
            ).wait()
            pltpu.async_copy(ob, dst, wsems[h % 2])
        return 0

    lax.fori_loop(0, F_PER_CORE, floop, 0)

    flast = fbase + F_PER_CORE - 1
    for h in range(NCH - 2, NCH):
        dst = out_hbm.at[flast, pl.ds(s * TILE_N + h * CH_N, CH_N)]
        pltpu.make_async_copy(obs[h % 2], dst, wsems[h % 2]).wait()


TCHUNK = 8
NCOL = TCHUNK * 2 * 128


def _tc_add_body(g_ref, seg_ref, comb_ref, o_ref):
    for tt in range(TCHUNK):
        a0 = comb_ref[0, 0, tt]
        a1 = comb_ref[1, 0, tt]
        a2 = comb_ref[2, 0, tt]
        for j2 in range(2):
            segrow = seg_ref[j2, tt, :][None, :]
            add = jnp.where(segrow == 0, a0,
                            jnp.where(segrow == 1, a1, a2))
            gg = g_ref[:, pl.ds((tt * 2 + j2) * 128, 128)]
            o_ref[tt, :, j2, :, :] = (gg + add).reshape(8, 8, 128)


def _tc_add(gathered, segP, combB):
    return pl.pallas_call(
        _tc_add_body,
        grid=(25, NS),
        in_specs=[
            pl.BlockSpec((EMBED, NCOL), lambda jt, tl: (0, tl * 25 + jt)),
            pl.BlockSpec((2, TCHUNK, 128), lambda jt, tl: (tl, jt, 0)),
            pl.BlockSpec((N_SEG, 1, TCHUNK, EMBED, 128),
                         lambda jt, tl: (0, jt, 0, 0, 0)),
        ],
        out_specs=pl.BlockSpec((TCHUNK, 8, 2, 8, 128),
                               lambda jt, tl: (jt, 0, tl, 0, 0)),
        out_shape=jax.ShapeDtypeStruct((SEQ_LEN, 8, NB // 128, 8, 128),
                                       jnp.float32),
    )(gathered, segP, combB)


def kernel(sequence, segment_label, token_table, segment_table):
    pe = _make_pe(512, EMBED)[:SEQ_LEN]
    comb3 = segment_table[:, None, :] + pe[None, :, :]
    combB = jnp.broadcast_to(
        comb3.reshape(N_SEG, 25, TCHUNK, EMBED, 1),
        (N_SEG, 25, TCHUNK, EMBED, 128),
    )
    seqP = (sequence.T.astype(jnp.int32)
            .reshape(SEQ_LEN, NS, 2, 128)
            .transpose(1, 0, 2, 3)
            .reshape(NS, TILE_N))
    segP = (segment_label.T.astype(jnp.int32)
            .reshape(SEQ_LEN, NB // 128, 128)
            .transpose(1, 0, 2))
    tokT = token_table.T
    gathered = _sc_gather(seqP, tokT)
    out5 = _tc_add(gathered, segP, combB)
    return out5.transpose(2, 4, 0, 1, 3).reshape(NB, SEQ_LEN, EMBED)


# speedup vs baseline: 7.8526x; 7.8096x over previous
import functools

import jax
import jax.numpy as jnp
import numpy as np
from jax import lax
from jax.experimental import pallas as pl
from jax.experimental.pallas import tpu as pltpu
from jax.experimental.pallas import tpu_sc as plsc

VOCAB = 1000000
EMBED = 64
SEQ_LEN = 200
N_SEG = 3

NC = 2
NS = 16
NW = NC * NS

B_TOTAL = 4096 * SEQ_LEN
ROWS_PER_W = B_TOTAL // NW
CHUNK = 128
NCHUNKS = ROWS_PER_W // CHUNK
LANES = 16
CSL = EMBED // LANES
ROW_UNROLL = 4


def _make_pe(max_len, d):
    position = jnp.arange(max_len, dtype=jnp.float32)[:, None]
    div_term = jnp.exp(
        jnp.arange(0, d, 2, dtype=jnp.float32) * (-np.log(10000.0) / d)
    )
    pe = jnp.zeros((max_len, d), dtype=jnp.float32)
    pe = pe.at[:, 0::2].set(jnp.sin(position * div_term))
    pe = pe.at[:, 1::2].set(jnp.cos(position * div_term))
    return pe


@functools.partial(
    pl.kernel,
    out_type=jax.ShapeDtypeStruct((B_TOTAL, EMBED), jnp.float32),
    mesh=plsc.VectorSubcoreMesh(core_axis_name="c", subcore_axis_name="s"),
    scratch_types=[
        pltpu.VMEM((NCHUNKS, CHUNK), jnp.int32),
        pltpu.VMEM((NCHUNKS, CHUNK), jnp.int32),
        pltpu.VMEM((2, CHUNK, EMBED), jnp.float32),
        pltpu.VMEM((2, CHUNK, EMBED), jnp.float32),
        pltpu.VMEM((2, CHUNK, EMBED), jnp.float32),
        pltpu.SemaphoreType.DMA,
        pltpu.SemaphoreType.DMA,
        pltpu.SemaphoreType.DMA,
        pltpu.SemaphoreType.DMA,
        pltpu.SemaphoreType.DMA,
        pltpu.SemaphoreType.DMA,
    ],
    compiler_params=pltpu.CompilerParams(use_tc_tiling_on_sc=False),
)
def _sc_embed(seq_hbm, seg_hbm, tok_hbm, comb_hbm, out_hbm,
              sidx_v, cidx_v, tok_v, cmb_v, res_v,
              sem_t0, sem_t1, sem_c0, sem_c1, sem_w0, sem_w1):
    wid = lax.axis_index("s") * NC + lax.axis_index("c")
    gchunk0 = wid * NCHUNKS
    iota = lax.iota(jnp.int32, LANES)
    sem_t = (sem_t0, sem_t1)
    sem_c = (sem_c0, sem_c1)
    sem_w = (sem_w0, sem_w1)

    pltpu.sync_copy(seq_hbm.at[pl.ds(gchunk0, NCHUNKS)], sidx_v)
    pltpu.sync_copy(seg_hbm.at[pl.ds(gchunk0, NCHUNKS)], cidx_v)

    def cidx_body(c, _):
        for j in range(CHUNK // LANES):
            start = lax.rem(c * CHUNK + j * LANES, SEQ_LEN)
            traw = start + iota
            t = jnp.where(traw >= SEQ_LEN, traw - SEQ_LEN, traw)
            sl = pl.ds(j * LANES, LANES)
            cidx_v[c, sl] = cidx_v[c, sl] * SEQ_LEN + t
        return 0

    lax.fori_loop(0, NCHUNKS, cidx_body, 0)

    def gather(c, b):
        pltpu.async_copy(tok_hbm.at[sidx_v.at[c]], tok_v.at[b], sem_t[b])
        pltpu.async_copy(comb_hbm.at[cidx_v.at[c]], cmb_v.at[b], sem_c[b])

    gather(0, 0)
    gather(1, 1)

    def pair_body(g, _):
        for b in range(2):
            c = g * 2 + b
            pltpu.make_async_copy(tok_hbm.at[sidx_v.at[c]], tok_v.at[b], sem_t[b]).wait()
            pltpu.make_async_copy(comb_hbm.at[cidx_v.at[c]], cmb_v.at[b], sem_c[b]).wait()
            base = (gchunk0 + c) * CHUNK

            @pl.when(g >= 1)
            def _():
                prev = (gchunk0 + c - 2) * CHUNK
                pltpu.make_async_copy(
                    res_v.at[b], out_hbm.at[pl.ds(prev, CHUNK)], sem_w[b]
                ).wait()

            def add_body(r4, _):
                for rr in range(ROW_UNROLL):
                    for cc in range(CSL):
                        sl = pl.ds(cc * LANES, LANES)
                        res_v[b, r4 * ROW_UNROLL + rr, sl] = (
                            tok_v[b, r4 * ROW_UNROLL + rr, sl]
                            + cmb_v[b, r4 * ROW_UNROLL + rr, sl]
                        )
                return 0

            lax.fori_loop(0, CHUNK // ROW_UNROLL, add_body, 0)

            pltpu.async_copy(res_v.at[b], out_hbm.at[pl.ds(base, CHUNK)], sem_w[b])

            @pl.when(g < NCHUNKS // 2 - 1)
            def _():
                gather(c + 2, b)
        return 0

    lax.fori_loop(0, NCHUNKS // 2, pair_body, 0)

    for b in range(2):
        last = (gchunk0 + NCHUNKS - 2 + b) * CHUNK
        pltpu.make_async_copy(
            res_v.at[b], out_hbm.at[pl.ds(last, CHUNK)], sem_w[b]
        ).wait()


def kernel(sequence, segment_label, token_table, segment_table):
    nb, sl = sequence.shape
    pe = _make_pe(512, EMBED)[:sl]
    comb = (segment_table[:, None, :] + pe[None, :, :]).reshape(N_SEG * sl, EMBED)
    seq_flat = sequence.reshape(-1, CHUNK).astype(jnp.int32)
    seg_flat = segment_label.reshape(-1, CHUNK).astype(jnp.int32)
    out = _sc_embed(seq_flat, seg_flat, token_table, comb)
    return out.reshape(nb, sl, EMBED)
